# baseline (speedup 1.0000x reference)
import functools

import jax
import jax.numpy as jnp
from jax import lax
from jax.experimental import pallas as pl
from jax.experimental.pallas import tpu as pltpu
from jax.experimental.pallas import tpu_sc as plsc

IN_F = 4096
OUT_F = 4096
TOTAL = IN_F * OUT_F

NW = 32
R = TOTAL // NW
WIN = 8192
WBUF = WIN + 16
SCH = WIN // 128
ZCHUNK = 8192
NZDMA = R // ZCHUNK


def _select_scalar(buf, i, nchunks, zero):
    del nchunks, zero
    return buf[pl.ds(i, 16)][0]


def _scatter_body(flat_hbm, vals_hbm, starts_hbm, dvals_hbm, w_hbm,
                  sbuf, dbuf, zbuf, fbuf, vbuf, idx_st, val_st,
                  sem_z, sem_s, sem_ld):
    w = lax.axis_index("s") * 2 + lax.axis_index("c")
    base = pl.multiple_of(w * R, ZCHUNK)

    with jax.named_scope("sc_prep"):
        pltpu.sync_copy(starts_hbm, sbuf)
        pltpu.sync_copy(dvals_hbm, dbuf)

        @pl.loop(0, ZCHUNK // 16)
        def _zero_init(i):
            zbuf[pl.ds(i * 16, 16)] = jnp.zeros((16,), jnp.float32)

    with jax.named_scope("sc_zero_fire"):
        @pl.loop(0, NZDMA)
        def _zero_fire(i):
            pltpu.async_copy(zbuf,
                             w_hbm.at[pl.ds(base + i * ZCHUNK, ZCHUNK)],
                             sem_z)

        start = _select_scalar(sbuf, w, 3, jnp.int32(0))
        end = _select_scalar(sbuf, w + 1, 3, jnp.int32(0))
        dval = _select_scalar(dbuf, w, 2, jnp.float32(0))
        astart = jnp.bitwise_and(start, jnp.int32(-8))
        delta = start - astart
        nmacro = (end - start + WIN - 1) // WIN

    with jax.named_scope("sc_zero_drain"):
        @pl.loop(0, NZDMA)
        def _zero_drain(i):
            pltpu.make_async_copy(zbuf, w_hbm.at[pl.ds(base, ZCHUNK)],
                                  sem_z).wait()

    def _macro(m, carry):
        with jax.named_scope("sc_load"):
            off = pl.multiple_of(astart + m * WIN, 8)
            ld1 = pltpu.async_copy(flat_hbm.at[pl.ds(off, WBUF)], fbuf,
                                   sem_ld)
            ld2 = pltpu.async_copy(vals_hbm.at[pl.ds(off, WBUF)], vbuf,
                                   sem_ld)
            ld1.wait()
            ld2.wait()

        with jax.named_scope("sc_stage"):
            @pl.loop(0, WIN // 16)
            def _stage(j):
                t = j * 16
                fv = fbuf[pl.ds(delta + t, 16)]
                vv = vbuf[pl.ds(delta + t, 16)]
                g = (start + m * WIN + t
                     + lax.broadcasted_iota(jnp.int32, (16,), 0))
                mask = g < end
                gi = jnp.where(mask, fv, base)
                gv = jnp.where(mask, vv, dval)
                row = j // 8
                col = (j % 8) * 16
                idx_st[row, pl.ds(col, 16)] = gi
                val_st[row, pl.ds(col, 16)] = gv

        with jax.named_scope("sc_fire"):
            @pl.loop(0, SCH)
            def _fire(r):
                pltpu.async_copy(val_st.at[r], w_hbm.at[idx_st.at[r]], sem_s)

        with jax.named_scope("sc_drain"):
            @pl.loop(0, SCH)
            def _drain(r):
                pltpu.make_async_copy(val_st.at[r], w_hbm.at[idx_st.at[r]],
                                      sem_s).wait()

        return carry

    lax.fori_loop(0, nmacro, _macro, 0)


def _densify(flat_p, vals_p, starts, dvals):
    mesh = plsc.VectorSubcoreMesh(core_axis_name="c", subcore_axis_name="s")
    return pl.kernel(
        _scatter_body,
        out_type=jax.ShapeDtypeStruct((TOTAL,), jnp.float32),
        mesh=mesh,
        scratch_types=[
            pltpu.VMEM((64,), jnp.int32),
            pltpu.VMEM((48,), jnp.float32),
            pltpu.VMEM((ZCHUNK,), jnp.float32),
            pltpu.VMEM((WBUF,), jnp.int32),
            pltpu.VMEM((WBUF,), jnp.float32),
            pltpu.VMEM((SCH, 128), jnp.int32),
            pltpu.VMEM((SCH, 128), jnp.float32),
            pltpu.SemaphoreType.DMA,
            pltpu.SemaphoreType.DMA,
            pltpu.SemaphoreType.DMA,
        ],
    )(flat_p, vals_p, starts, dvals)


def _mm_body(x_ref, w_ref, b_ref, o_ref):
    acc = lax.dot_general(
        x_ref[...], w_ref[...],
        (((1,), (1,)), ((), ())),
        preferred_element_type=jnp.float32)
    o_ref[...] = acc + b_ref[...][None, :]


def _matmul(x, w, bias, batch):
    nb = 512
    return pl.pallas_call(
        _mm_body,
        grid=(OUT_F // nb,),
        in_specs=[
            pl.BlockSpec((batch, IN_F), lambda j: (0, 0)),
            pl.BlockSpec((nb, IN_F), lambda j: (j, 0)),
            pl.BlockSpec((nb,), lambda j: (j,)),
        ],
        out_specs=pl.BlockSpec((batch, nb), lambda j: (0, j)),
        out_shape=jax.ShapeDtypeStruct((batch, OUT_F), jnp.float32),
    )(x, w, bias)


def kernel(x, sparse_indices, sparse_values, bias):
    orig_shape = x.shape
    x2d = x.reshape(-1, IN_F)
    batch = x2d.shape[0]

    nnz = sparse_values.shape[0]
    flat = sparse_indices[0] * IN_F + sparse_indices[1]
    padn = -(-(nnz + 2 * WIN) // 16) * 16
    pad = padn - nnz
    flat_p = jnp.concatenate([flat, jnp.zeros((pad,), flat.dtype)])
    vals_p = jnp.concatenate(
        [sparse_values, jnp.zeros((pad,), sparse_values.dtype)])
    bounds = jnp.arange(33, dtype=flat.dtype) * R
    starts = jnp.searchsorted(flat, bounds).astype(jnp.int32)
    starts_p = jnp.concatenate([starts, jnp.zeros((31,), jnp.int32)])
    own = jnp.arange(32, dtype=jnp.int32) * R
    s32 = starts[:32]
    dvals = jnp.where(flat_p[s32] == own, vals_p[s32], 0.0)
    dvals = jnp.concatenate([dvals, jnp.zeros((16,), jnp.float32)])

    w_flat = _densify(flat_p, vals_p, starts_p, dvals)
    w = w_flat.reshape(OUT_F, IN_F)
    y = _matmul(x2d, w, bias, batch)
    return y.reshape(*orig_shape[:-1], OUT_F).astype(x.dtype)


# speedup vs baseline: 9.8055x; 9.8055x over previous
import functools

import jax
import jax.numpy as jnp
from jax import lax
from jax.experimental import pallas as pl
from jax.experimental.pallas import tpu as pltpu
from jax.experimental.pallas import tpu_sc as plsc

IN_F = 4096
OUT_F = 4096
TOTAL = IN_F * OUT_F

NW = 32
CSZ = 65536
NCH = TOTAL // CSZ
NSUB = NCH // NW
WIN = 8192
WBUF = WIN + 16
SBUF = ((NCH + 1 + 31) // 16) * 16


def _sel(buf, i):
    return buf[pl.ds(i, 16)][0]


def _scatter_body(flat_hbm, vals_hbm, starts_hbm, w_hbm,
                  sbuf, dense, fwin, vwin, sem_ld, sem_o):
    w = lax.axis_index("s") * 2 + lax.axis_index("c")

    pltpu.sync_copy(starts_hbm, sbuf)

    @pl.loop(0, CSZ // 16)
    def _zero_init(i):
        dense[pl.ds(i * 16, 16)] = jnp.zeros((16,), jnp.float32)

    def _windows(c, start, end, value_of):
        astart = jnp.bitwise_and(start, jnp.int32(-8))
        delta = start - astart
        cbase = c * CSZ
        nwin = (end - start + WIN - 1) // WIN

        def _win(m, carry):
            off = pl.multiple_of(astart + m * WIN, 8)
            ld1 = pltpu.async_copy(flat_hbm.at[pl.ds(off, WBUF)], fwin,
                                   sem_ld)
            ld2 = pltpu.async_copy(vals_hbm.at[pl.ds(off, WBUF)], vwin,
                                   sem_ld)
            ld1.wait()
            ld2.wait()
            rem = end - start - m * WIN
            n_j = (jnp.minimum(rem, WIN) + 15) // 16
            wbase = start + m * WIN

            @pl.loop(0, n_j)
            def _scat(j):
                t = j * 16
                fv = fwin[pl.ds(delta + t, 16)]
                vv = vwin[pl.ds(delta + t, 16)]
                g = wbase + t + lax.broadcasted_iota(jnp.int32, (16,), 0)
                mask = g < end
                lidx = fv - cbase
                plsc.store_scatter(dense, [lidx], value_of(vv), mask=mask)

            return carry

        lax.fori_loop(0, nwin, _win, 0)

    @pl.loop(0, NSUB)
    def _sub(s):
        c = w * NSUB + s
        start = _sel(sbuf, c)
        end = _sel(sbuf, c + 1)
        with jax.named_scope("sc_scatter"):
            _windows(c, start, end, lambda v: v)
        with jax.named_scope("sc_dma_out"):
            cb = pl.multiple_of(c * CSZ, CSZ)
            pltpu.async_copy(dense, w_hbm.at[pl.ds(cb, CSZ)], sem_o).wait()
        with jax.named_scope("sc_clean"):
            _windows(c, start, end,
                     lambda v: jnp.zeros((16,), jnp.float32))


def _densify(flat_p, vals_p, starts):
    mesh = plsc.VectorSubcoreMesh(core_axis_name="c", subcore_axis_name="s")
    return pl.kernel(
        _scatter_body,
        out_type=jax.ShapeDtypeStruct((TOTAL,), jnp.float32),
        mesh=mesh,
        compiler_params=pltpu.CompilerParams(needs_layout_passes=False),
        scratch_types=[
            pltpu.VMEM((SBUF,), jnp.int32),
            pltpu.VMEM((CSZ,), jnp.float32),
            pltpu.VMEM((WBUF,), jnp.int32),
            pltpu.VMEM((WBUF,), jnp.float32),
            pltpu.SemaphoreType.DMA,
            pltpu.SemaphoreType.DMA,
        ],
    )(flat_p, vals_p, starts)


def _mm_body(x_ref, w_ref, b_ref, o_ref):
    acc = lax.dot_general(
        x_ref[...], w_ref[...],
        (((1,), (1,)), ((), ())),
        preferred_element_type=jnp.float32)
    o_ref[...] = acc + b_ref[...][None, :]


def _matmul(x, w, bias, batch):
    nb = 512
    return pl.pallas_call(
        _mm_body,
        grid=(OUT_F // nb,),
        in_specs=[
            pl.BlockSpec((batch, IN_F), lambda j: (0, 0)),
            pl.BlockSpec((nb, IN_F), lambda j: (j, 0)),
            pl.BlockSpec((nb,), lambda j: (j,)),
        ],
        out_specs=pl.BlockSpec((batch, nb), lambda j: (0, j)),
        out_shape=jax.ShapeDtypeStruct((batch, OUT_F), jnp.float32),
    )(x, w, bias)


def kernel(x, sparse_indices, sparse_values, bias):
    orig_shape = x.shape
    x2d = x.reshape(-1, IN_F)
    batch = x2d.shape[0]

    nnz = sparse_values.shape[0]
    flat = sparse_indices[0] * IN_F + sparse_indices[1]
    padn = -(-(nnz + 2 * WIN) // 16) * 16
    pad = padn - nnz
    flat_p = jnp.concatenate([flat, jnp.zeros((pad,), flat.dtype)])
    vals_p = jnp.concatenate(
        [sparse_values, jnp.zeros((pad,), sparse_values.dtype)])
    bounds = jnp.arange(NCH + 1, dtype=flat.dtype) * CSZ
    starts = jnp.searchsorted(flat, bounds).astype(jnp.int32)
    starts_p = jnp.concatenate(
        [starts, jnp.zeros((SBUF - NCH - 1,), jnp.int32)])

    w_flat = _densify(flat_p, vals_p, starts_p)
    w = w_flat.reshape(OUT_F, IN_F)
    y = _matmul(x2d, w, bias, batch)
    return y.reshape(*orig_shape[:-1], OUT_F).astype(x.dtype)


# speedup vs baseline: 12.6987x; 1.2951x over previous
import functools

import jax
import jax.numpy as jnp
from jax import lax
from jax.experimental import pallas as pl
from jax.experimental.pallas import tpu as pltpu
from jax.experimental.pallas import tpu_sc as plsc

IN_F = 4096
OUT_F = 4096
TOTAL = IN_F * OUT_F

NW = 32
CSZ = 65536
NCH = TOTAL // CSZ
NSUB = NCH // NW
WIN = 8192
WBUF = WIN + 16
SBUF = ((NCH + 1 + 31) // 16) * 16


def _sel(buf, i):
    return buf[pl.ds(i, 16)][0]


def _scatter_body(flat_hbm, vals_hbm, starts_hbm, w_hbm,
                  sbuf, dense, fwin, vwin, sem_ld, sem_o):
    w = lax.axis_index("s") * 2 + lax.axis_index("c")

    pltpu.sync_copy(starts_hbm, sbuf)

    @pl.loop(0, CSZ // 16)
    def _zero_init(i):
        dense[pl.ds(i * 16, 16)] = jnp.zeros((16,), jnp.float32)

    def _windows(c, start, end, value_of):
        astart = jnp.bitwise_and(start, jnp.int32(-8))
        delta = start - astart
        cbase = c * CSZ
        nwin = (end - start + WIN - 1) // WIN

        def _win(m, carry):
            off = pl.multiple_of(astart + m * WIN, 8)
            ld1 = pltpu.async_copy(flat_hbm.at[pl.ds(off, WBUF)], fwin,
                                   sem_ld)
            ld2 = pltpu.async_copy(vals_hbm.at[pl.ds(off, WBUF)], vwin,
                                   sem_ld)
            ld1.wait()
            ld2.wait()
            rem = end - start - m * WIN
            n_j = (jnp.minimum(rem, WIN) + 15) // 16
            wbase = start + m * WIN

            @pl.loop(0, n_j)
            def _scat(j):
                t = j * 16
                fv = fwin[pl.ds(delta + t, 16)]
                vv = vwin[pl.ds(delta + t, 16)]
                g = wbase + t + lax.broadcasted_iota(jnp.int32, (16,), 0)
                mask = g < end
                lidx = fv - cbase
                plsc.store_scatter(dense, [lidx], value_of(vv), mask=mask)

            return carry

        lax.fori_loop(0, nwin, _win, 0)

    @pl.loop(0, NSUB)
    def _sub(s):
        c = w * NSUB + s
        start = _sel(sbuf, c)
        end = _sel(sbuf, c + 1)
        with jax.named_scope("sc_scatter"):
            _windows(c, start, end, lambda v: v)
        with jax.named_scope("sc_dma_out"):
            cb = pl.multiple_of(c * CSZ, CSZ)
            pltpu.async_copy(dense, w_hbm.at[pl.ds(cb, CSZ)], sem_o).wait()
        with jax.named_scope("sc_clean"):
            _windows(c, start, end,
                     lambda v: jnp.zeros((16,), jnp.float32))


def _densify(flat_p, vals_p, starts):
    mesh = plsc.VectorSubcoreMesh(core_axis_name="c", subcore_axis_name="s")
    return pl.kernel(
        _scatter_body,
        out_type=jax.ShapeDtypeStruct((TOTAL,), jnp.float32),
        mesh=mesh,
        compiler_params=pltpu.CompilerParams(needs_layout_passes=False),
        scratch_types=[
            pltpu.VMEM((SBUF,), jnp.int32),
            pltpu.VMEM((CSZ,), jnp.float32),
            pltpu.VMEM((WBUF,), jnp.int32),
            pltpu.VMEM((WBUF,), jnp.float32),
            pltpu.SemaphoreType.DMA,
            pltpu.SemaphoreType.DMA,
        ],
    )(flat_p, vals_p, starts)


def _mm_body(x_ref, w_ref, b_ref, o_ref):
    acc = lax.dot_general(
        x_ref[...], w_ref[...],
        (((1,), (1,)), ((), ())),
        preferred_element_type=jnp.float32)
    o_ref[...] = acc + b_ref[...][None, :]


def _matmul(x, w, bias, batch):
    nb = 512
    return pl.pallas_call(
        _mm_body,
        grid=(OUT_F // nb,),
        in_specs=[
            pl.BlockSpec((batch, IN_F), lambda j: (0, 0)),
            pl.BlockSpec((nb, IN_F), lambda j: (j, 0)),
            pl.BlockSpec((nb,), lambda j: (j,)),
        ],
        out_specs=pl.BlockSpec((batch, nb), lambda j: (0, j)),
        out_shape=jax.ShapeDtypeStruct((batch, OUT_F), jnp.float32),
    )(x, w, bias)


def kernel(x, sparse_indices, sparse_values, bias):
    orig_shape = x.shape
    x2d = x.reshape(-1, IN_F)
    batch = x2d.shape[0]

    nnz = sparse_values.shape[0]
    flat = sparse_indices[0] * IN_F + sparse_indices[1]
    padn = -(-(nnz + 2 * WIN) // 16) * 16
    pad = padn - nnz
    flat_p = jnp.concatenate([flat, jnp.zeros((pad,), flat.dtype)])
    vals_p = jnp.concatenate(
        [sparse_values, jnp.zeros((pad,), sparse_values.dtype)])
    S = 512
    ns = -(-nnz // S)
    pad_hi = jnp.full((ns * S - nnz,), jnp.iinfo(jnp.int32).max, jnp.int32)
    flat_hi = jnp.concatenate([flat, pad_hi])
    sample = flat_hi[::S]
    bounds = jnp.arange(NCH + 1, dtype=flat.dtype) * CSZ
    coarse = jnp.searchsorted(sample, bounds).astype(jnp.int32)
    base = jnp.maximum(coarse - 1, 0) * S
    wins = flat_hi[base[:, None] + jnp.arange(S, dtype=jnp.int32)[None, :]]
    cnts = jnp.sum(wins < bounds[:, None], axis=1).astype(jnp.int32)
    starts = base + cnts
    starts_p = jnp.concatenate(
        [starts, jnp.zeros((SBUF - NCH - 1,), jnp.int32)])

    w_flat = _densify(flat_p, vals_p, starts_p)
    w = w_flat.reshape(OUT_F, IN_F)
    y = _matmul(x2d, w, bias, batch)
    return y.reshape(*orig_shape[:-1], OUT_F).astype(x.dtype)
